# baseline (speedup 1.0000x reference)
import functools
import math

import jax
import jax.numpy as jnp
from jax import lax
from jax.experimental import pallas as pl
from jax.experimental.pallas import tpu as pltpu
from jax.experimental.pallas import tpu_sc as plsc

N = 10000
E = 320000
D = 128
NUM_HE = 10000
ALPHA = 0.1
LAMDA = 0.5

NC = 2
NS = 16
NW = NC * NS
K = 80
C_STG = (E // NS) // K
C_CNT = (E // NW) // K
HEP = 10240
RPT = HEP // NS
DH = D // NC

_MESH = plsc.VectorSubcoreMesh(core_axis_name="c", subcore_axis_name="s")


def _make_sc_stage():
  scratch = [
      pltpu.VMEM((C_STG, K), jnp.int32),
      pltpu.VMEM((C_STG, K), jnp.int32),
      pltpu.VMEM((K, DH), jnp.float32),
      pltpu.VMEM((K, DH), jnp.float32),
      pltpu.VMEM_SHARED((HEP, DH), jnp.float32),
      pltpu.SemaphoreType.DMA,
      pltpu.SemaphoreType.DMA,
  ]

  @functools.partial(
      pl.kernel,
      out_type=jax.ShapeDtypeStruct((NC, HEP, DH), jnp.float32),
      mesh=_MESH, scratch_types=tuple(scratch),
      compiler_params=pltpu.CompilerParams(needs_layout_passes=False,
                                           use_tc_tiling_on_sc=False))
  def stage(src, gidx3, sidx3, zdh, out,
            gidx_v, sidx_v, rows_a, rows_b, acc, sem_a, sem_b):
    c = lax.axis_index("c")
    s = lax.axis_index("s")
    row0 = s * RPT

    pltpu.sync_copy(zdh, acc.at[pl.ds(row0, RPT)])
    pltpu.sync_copy(gidx3.at[s], gidx_v)
    pltpu.sync_copy(sidx3.at[s], sidx_v)
    plsc.subcore_barrier()

    srcc = src.at[c]

    @pl.loop(0, C_STG // 2)
    def _pair(t):
      j0 = 2 * t
      cp_a = pltpu.async_copy(srcc.at[gidx_v.at[j0]], rows_a, sem_a)
      cp_b = pltpu.async_copy(srcc.at[gidx_v.at[j0 + 1]], rows_b, sem_b)
      cp_a.wait()
      pltpu.sync_copy(rows_a, acc.at[sidx_v.at[j0]], add=True)
      cp_b.wait()
      pltpu.sync_copy(rows_b, acc.at[sidx_v.at[j0 + 1]], add=True)

    plsc.subcore_barrier()
    pltpu.sync_copy(acc.at[pl.ds(row0, RPT)], out.at[c].at[pl.ds(row0, RPT)])

  return stage


def _make_sc_counts():
  scratch = [
      pltpu.VMEM((C_CNT, K), jnp.int32),
      pltpu.VMEM((C_CNT, K), jnp.int32),
      pltpu.VMEM((HEP,), jnp.float32),
      pltpu.VMEM((HEP,), jnp.float32),
  ]
  out_sds = jax.ShapeDtypeStruct((NC, NS, HEP), jnp.float32)

  @functools.partial(
      pl.kernel, out_type=(out_sds, out_sds),
      mesh=_MESH, scratch_types=tuple(scratch),
      compiler_params=pltpu.CompilerParams(needs_layout_passes=False))
  def counts(vidx3, eidx3, out_cv, out_ce, vidx_v, eidx_v, cv_loc, ce_loc):
    c = lax.axis_index("c")
    s = lax.axis_index("s")
    w = c * NS + s
    pltpu.sync_copy(vidx3.at[w], vidx_v)
    pltpu.sync_copy(eidx3.at[w], eidx_v)
    z16v = jnp.zeros((16,), jnp.float32)

    @pl.loop(0, HEP // 16)
    def _zero(i):
      cv_loc[pl.ds(i * 16, 16)] = z16v
      ce_loc[pl.ds(i * 16, 16)] = z16v

    ones16 = jnp.ones((16,), jnp.float32)

    @pl.loop(0, C_CNT)
    def _chunk(j):
      for m in range(K // 16):
        vi = vidx_v[j, pl.ds(m * 16, 16)]
        ei = eidx_v[j, pl.ds(m * 16, 16)]
        plsc.addupdate_scatter(cv_loc, [vi], ones16)
        plsc.addupdate_scatter(ce_loc, [ei], ones16)

    pltpu.sync_copy(cv_loc, out_cv.at[c].at[s])
    pltpu.sync_copy(ce_loc, out_ce.at[c].at[s])

  return counts


_sc_stage = _make_sc_stage()
_sc_counts = _make_sc_counts()

BM = 1000


def _specs(shapes):
  specs = []
  for shp in shapes:
    if len(shp) == 3:
      specs.append(pl.BlockSpec((shp[0], BM, shp[2]), lambda i: (0, i, 0)))
    elif shp[0] in (N, NUM_HE):
      specs.append(pl.BlockSpec((BM, shp[1]), lambda i: (i, 0)))
    else:
      specs.append(pl.BlockSpec(shp, lambda i: (0,) * len(shp)))
  return specs


def _tc_call(body, i_shapes, o_shapes):
  out_shape = [jax.ShapeDtypeStruct(shp, jnp.float32) for shp in o_shapes]
  return pl.pallas_call(
      body,
      grid=(N // BM,),
      in_specs=_specs(i_shapes),
      out_specs=_specs(o_shapes),
      out_shape=out_shape,
  )


def _dot_t(a, w):
  return lax.dot_general(a, w, (((1,), (1,)), ((), ())),
                         preferred_element_type=jnp.float32)


def _split_store(o_ref, h):
  o_ref[0] = h[:, :DH]
  o_ref[1] = h[:, DH:]


def _in_proj_body(x_ref, w_ref, b_ref, o_ref, os_ref):
  h = jax.nn.relu(_dot_t(x_ref[...], w_ref[...]) + b_ref[...])
  o_ref[...] = h
  _split_store(os_ref, h)


def _recip_cnt(cc):
  cnt = jnp.sum(cc, axis=0)
  return 1.0 / jnp.maximum(cnt, 1.0)


def _mean_body(p_ref, c_ref, o_ref, os_ref):
  p = p_ref[...]
  xe = jnp.concatenate([p[0], p[1]], axis=-1) * _recip_cnt(c_ref[...])
  o_ref[...] = xe
  _split_store(os_ref, xe)


def _update_body(alpha, beta, p_ref, c_ref, h0_ref, w_ref, os_ref):
  p = p_ref[...]
  xv = jnp.concatenate([p[0], p[1]], axis=-1) * _recip_cnt(c_ref[...])
  xi = (1.0 - alpha) * xv + alpha * h0_ref[...]
  h = jax.nn.relu((1.0 - beta) * xi + beta * _dot_t(xi, w_ref[...]))
  _split_store(os_ref, h)


def _update_final_body(alpha, beta, p_ref, c_ref, h0_ref, w_ref,
                       wl_ref, bl_ref, o_ref):
  p = p_ref[...]
  xv = jnp.concatenate([p[0], p[1]], axis=-1) * _recip_cnt(c_ref[...])
  xi = (1.0 - alpha) * xv + alpha * h0_ref[...]
  h = jax.nn.relu((1.0 - beta) * xi + beta * _dot_t(xi, w_ref[...]))
  o_ref[...] = _dot_t(h, wl_ref[...]) + bl_ref[...]


def kernel(x, hyperedge_index, W0, b0, Wc0, Wc1, Wlast, blast):
  vertex = hyperedge_index[0]
  edges = hyperedge_index[1]
  v3s = vertex.reshape(NS, C_STG, K)
  e3s = edges.reshape(NS, C_STG, K)
  v3c = vertex.reshape(NW, C_CNT, K)
  e3c = edges.reshape(NW, C_CNT, K)
  zdh = jnp.zeros((RPT, DH), jnp.float32)
  b0r = b0.reshape(1, 128)
  blr = blast.reshape(1, 128)

  beta0 = math.log(LAMDA / 1.0 + 1.0)
  beta1 = math.log(LAMDA / 2.0 + 1.0)

  pcv, pce = _sc_counts(v3c, e3c)
  pce_r = pce.reshape(NW, HEP, 1)
  pcv_r = pcv.reshape(NW, HEP, 1)

  h0, h0s = _tc_call(_in_proj_body, [(N, D), (D, D), (1, D)],
                     [(N, D), (NC, N, DH)])(x, W0, b0r)

  pe1 = _sc_stage(h0s, v3s, e3s, zdh)
  _, xe1s = _tc_call(_mean_body, [(NC, HEP, DH), (NW, HEP, 1)],
                     [(N, D), (NC, N, DH)])(pe1, pce_r)
  pv1 = _sc_stage(xe1s, e3s, v3s, zdh)
  (h1s,) = _tc_call(
      functools.partial(_update_body, ALPHA, beta0),
      [(NC, HEP, DH), (NW, HEP, 1), (N, D), (D, D)],
      [(NC, N, DH)])(pv1, pcv_r, h0, Wc0)

  pe2 = _sc_stage(h1s, v3s, e3s, zdh)
  xe2, xe2s = _tc_call(_mean_body, [(NC, HEP, DH), (NW, HEP, 1)],
                       [(N, D), (NC, N, DH)])(pe2, pce_r)
  pv2 = _sc_stage(xe2s, e3s, v3s, zdh)
  (out,) = _tc_call(
      functools.partial(_update_final_body, ALPHA, beta1),
      [(NC, HEP, DH), (NW, HEP, 1), (N, D), (D, D), (D, D), (1, D)],
      [(N, D)])(pv2, pcv_r, h0, Wc1, Wlast, blr)

  return (out, xe2)


# speedup vs baseline: 1.0560x; 1.0560x over previous
import functools
import math

import jax
import jax.numpy as jnp
from jax import lax
from jax.experimental import pallas as pl
from jax.experimental.pallas import tpu as pltpu
from jax.experimental.pallas import tpu_sc as plsc

N = 10000
E = 320000
D = 128
NUM_HE = 10000
ALPHA = 0.1
LAMDA = 0.5

NC = 2
NS = 16
NW = NC * NS
EPT = E // NW
K = 80
C = EPT // K
HEP = 10240
RPT = HEP // NS

_MESH = plsc.VectorSubcoreMesh(core_axis_name="c", subcore_axis_name="s")


def _make_sc_stage():
  scratch = [
      pltpu.VMEM((C, K), jnp.int32),
      pltpu.VMEM((C, K), jnp.int32),
      pltpu.VMEM((K, 128), jnp.float32),
      pltpu.VMEM_SHARED((HEP, 128), jnp.float32),
      pltpu.SemaphoreType.DMA,
  ]

  @functools.partial(
      pl.kernel,
      out_type=jax.ShapeDtypeStruct((NC, HEP, 128), jnp.float32),
      mesh=_MESH, scratch_types=tuple(scratch),
      compiler_params=pltpu.CompilerParams(needs_layout_passes=False))
  def stage(src, gidx3, sidx3, z128, out,
            gidx_v, sidx_v, rows_v, acc, sem):
    c = lax.axis_index("c")
    s = lax.axis_index("s")
    w = c * NS + s
    row0 = s * RPT

    pltpu.sync_copy(z128, acc.at[pl.ds(row0, RPT)])
    pltpu.sync_copy(gidx3.at[w], gidx_v)
    pltpu.sync_copy(sidx3.at[w], sidx_v)
    plsc.subcore_barrier()

    @pl.loop(0, C)
    def _chunk(j):
      pltpu.async_copy(src.at[gidx_v.at[j]], rows_v, sem).wait()
      pltpu.sync_copy(rows_v, acc.at[sidx_v.at[j]], add=True)

    plsc.subcore_barrier()
    pltpu.sync_copy(acc.at[pl.ds(row0, RPT)], out.at[c].at[pl.ds(row0, RPT)])

  return stage


def _make_sc_counts():
  scratch = [
      pltpu.VMEM((C, K), jnp.int32),
      pltpu.VMEM((C, K), jnp.int32),
      pltpu.VMEM((HEP,), jnp.float32),
      pltpu.VMEM((HEP,), jnp.float32),
  ]
  out_sds = jax.ShapeDtypeStruct((NC, NS, HEP), jnp.float32)

  @functools.partial(
      pl.kernel, out_type=(out_sds, out_sds),
      mesh=_MESH, scratch_types=tuple(scratch),
      compiler_params=pltpu.CompilerParams(needs_layout_passes=False))
  def counts(vidx3, eidx3, out_cv, out_ce, vidx_v, eidx_v, cv_loc, ce_loc):
    c = lax.axis_index("c")
    s = lax.axis_index("s")
    w = c * NS + s
    pltpu.sync_copy(vidx3.at[w], vidx_v)
    pltpu.sync_copy(eidx3.at[w], eidx_v)
    z16v = jnp.zeros((16,), jnp.float32)

    @pl.loop(0, HEP // 16)
    def _zero(i):
      cv_loc[pl.ds(i * 16, 16)] = z16v
      ce_loc[pl.ds(i * 16, 16)] = z16v

    ones16 = jnp.ones((16,), jnp.float32)

    @pl.loop(0, C)
    def _chunk(j):
      for m in range(K // 16):
        vi = vidx_v[j, pl.ds(m * 16, 16)]
        ei = eidx_v[j, pl.ds(m * 16, 16)]
        plsc.addupdate_scatter(cv_loc, [vi], ones16)
        plsc.addupdate_scatter(ce_loc, [ei], ones16)

    pltpu.sync_copy(cv_loc, out_cv.at[c].at[s])
    pltpu.sync_copy(ce_loc, out_ce.at[c].at[s])

  return counts


_sc_stage = _make_sc_stage()
_sc_counts = _make_sc_counts()

BM = 1000


def _row_specs(i_shapes):
  specs = []
  for shp in i_shapes:
    if len(shp) == 3:
      specs.append(pl.BlockSpec((shp[0], BM, shp[2]), lambda i: (0, i, 0)))
    elif shp[0] in (N, NUM_HE):
      specs.append(pl.BlockSpec((BM, shp[1]), lambda i: (i, 0)))
    else:
      specs.append(pl.BlockSpec(shp, lambda i: (0,) * len(shp)))
  return specs


def _tc_call(body, i_shapes):
  return pl.pallas_call(
      body,
      grid=(N // BM,),
      in_specs=_row_specs(i_shapes),
      out_specs=pl.BlockSpec((BM, 128), lambda i: (i, 0)),
      out_shape=jax.ShapeDtypeStruct((N, 128), jnp.float32),
  )


def _dot_t(a, w):
  return lax.dot_general(a, w, (((1,), (1,)), ((), ())),
                         preferred_element_type=jnp.float32)


def _in_proj_body(x_ref, w_ref, b_ref, o_ref):
  o_ref[...] = jax.nn.relu(_dot_t(x_ref[...], w_ref[...]) + b_ref[...])


def _recip_cnt(cc):
  cnt = jnp.sum(cc, axis=0)
  return 1.0 / jnp.maximum(cnt, 1.0)


def _mean_body(p_ref, c_ref, o_ref):
  p = p_ref[...]
  o_ref[...] = (p[0] + p[1]) * _recip_cnt(c_ref[...])


def _update_body(alpha, beta, p_ref, c_ref, h0_ref, w_ref, o_ref):
  p = p_ref[...]
  xv = (p[0] + p[1]) * _recip_cnt(c_ref[...])
  xi = (1.0 - alpha) * xv + alpha * h0_ref[...]
  o_ref[...] = jax.nn.relu((1.0 - beta) * xi + beta * _dot_t(xi, w_ref[...]))


def _update_final_body(alpha, beta, p_ref, c_ref, h0_ref, w_ref,
                       wl_ref, bl_ref, o_ref):
  p = p_ref[...]
  xv = (p[0] + p[1]) * _recip_cnt(c_ref[...])
  xi = (1.0 - alpha) * xv + alpha * h0_ref[...]
  h = jax.nn.relu((1.0 - beta) * xi + beta * _dot_t(xi, w_ref[...]))
  o_ref[...] = _dot_t(h, wl_ref[...]) + bl_ref[...]


def kernel(x, hyperedge_index, W0, b0, Wc0, Wc1, Wlast, blast):
  vertex = hyperedge_index[0]
  edges = hyperedge_index[1]
  v3 = vertex.reshape(NW, C, K)
  e3 = edges.reshape(NW, C, K)
  z128 = jnp.zeros((RPT, 128), jnp.float32)
  b0r = b0.reshape(1, 128)
  blr = blast.reshape(1, 128)

  beta0 = math.log(LAMDA / 1.0 + 1.0)
  beta1 = math.log(LAMDA / 2.0 + 1.0)

  pcv, pce = _sc_counts(v3, e3)
  pce_r = pce.reshape(NW, HEP, 1)
  pcv_r = pcv.reshape(NW, HEP, 1)

  h0 = _tc_call(_in_proj_body, [(N, D), (D, D), (1, D)])(x, W0, b0r)

  pe1 = _sc_stage(h0, v3, e3, z128)
  xe1 = _tc_call(_mean_body, [(NC, HEP, 128), (NW, HEP, 1)])(pe1, pce_r)
  pv1 = _sc_stage(xe1, e3, v3, z128)
  h1 = _tc_call(
      functools.partial(_update_body, ALPHA, beta0),
      [(NC, HEP, 128), (NW, HEP, 1), (N, 128), (D, D)])(pv1, pcv_r, h0, Wc0)

  pe2 = _sc_stage(h1, v3, e3, z128)
  xe2 = _tc_call(_mean_body, [(NC, HEP, 128), (NW, HEP, 1)])(pe2, pce_r)
  pv2 = _sc_stage(xe2, e3, v3, z128)
  out = _tc_call(
      functools.partial(_update_final_body, ALPHA, beta1),
      [(NC, HEP, 128), (NW, HEP, 1), (N, 128), (D, D), (D, D), (1, D)])(
          pv2, pcv_r, h0, Wc1, Wlast, blr)

  return (out, xe2)
